# baseline (speedup 1.0000x reference)
import functools

import jax
import jax.numpy as jnp
from jax import lax
from jax.experimental import pallas as pl
from jax.experimental.pallas import tpu as pltpu
from jax.experimental.pallas import tpu_sc as plsc

_N = 10000
_E = 320000
_H = 128
_R = 8
_R_MAX = 5.0
_SHARP = 6.0


_NW = 32
_NHALF = 5056
_NACC = 5120
_ZR = 320


@functools.lru_cache(maxsize=None)
def _sc_gather_kernel(T, W, rows, CH):
    per_w = rows // CH // _NW
    mesh = plsc.VectorSubcoreMesh(core_axis_name="c", subcore_axis_name="s")

    @functools.partial(
        pl.kernel,
        out_type=jax.ShapeDtypeStruct((rows, W), jnp.float32),
        mesh=mesh,
        scratch_types=[
            pltpu.VMEM((CH,), jnp.int32),
            pltpu.VMEM((CH,), jnp.int32),
            pltpu.VMEM((CH, W), jnp.float32),
            pltpu.VMEM((CH, W), jnp.float32),
            pltpu.SemaphoreType.DMA,
            pltpu.SemaphoreType.DMA,
            pltpu.SemaphoreType.DMA,
            pltpu.SemaphoreType.DMA,
        ],
    )
    def k(table_hbm, idx_hbm, out_hbm, idx0, idx1, rows0, rows1,
          g0, g1, w0, w1):
        wid = lax.axis_index("s") * 2 + lax.axis_index("c")
        idx_b = (idx0, idx1)
        rows_b = (rows0, rows1)
        gsem = (g0, g1)
        wsem = (w0, w1)

        def base_of(c):
            return (c * _NW + wid) * CH

        if per_w == 1:
            pltpu.sync_copy(idx_hbm.at[pl.ds(base_of(0), CH)], idx0)
            pltpu.async_copy(table_hbm.at[idx0], rows0, g0).wait()
            pltpu.sync_copy(rows0, out_hbm.at[pl.ds(base_of(0), CH)])
            return

        for b in range(2):
            pltpu.sync_copy(idx_hbm.at[pl.ds(base_of(b), CH)], idx_b[b])
            pltpu.async_copy(table_hbm.at[idx_b[b]], rows_b[b], gsem[b])

        def wait_gather(b):
            pltpu.make_async_copy(
                table_hbm.at[idx_b[b]], rows_b[b], gsem[b]).wait()

        def wait_write(b):
            pltpu.make_async_copy(
                rows_b[b], out_hbm.at[pl.ds(0, CH)], wsem[b]).wait()

        def body(i, carry):
            for b in range(2):
                cc = 2 + 2 * i + b
                wait_gather(b)
                pltpu.async_copy(
                    rows_b[b], out_hbm.at[pl.ds(base_of(cc - 2), CH)], wsem[b])
                pltpu.sync_copy(idx_hbm.at[pl.ds(base_of(cc), CH)], idx_b[b])
                wait_write(b)
                pltpu.async_copy(table_hbm.at[idx_b[b]], rows_b[b], gsem[b])
            return carry

        lax.fori_loop(0, (per_w - 2) // 2, body, 0)
        for b in range(2):
            cc = per_w - 2 + b
            wait_gather(b)
            pltpu.sync_copy(rows_b[b], out_hbm.at[pl.ds(base_of(cc), CH)])

    return k


def _sc_gather(table, idx, CH):
    return _sc_gather_kernel(table.shape[0], table.shape[1], idx.shape[0],
                             CH)(table, idx)


_RQ = 1264
_RACC = 1280
_RSTR = 80
_NPASS = 4


@functools.lru_cache(maxsize=None)
def _sc_scatter128_kernel(nsub, rows, CH):
    per_sc = rows // CH // 16
    mesh = plsc.VectorSubcoreMesh(core_axis_name="c", subcore_axis_name="s")
    outs = tuple(jax.ShapeDtypeStruct((2 * _NPASS, _RACC, 128), jnp.float32)
                 for _ in range(nsub))

    @functools.partial(
        pl.kernel,
        out_type=outs,
        mesh=mesh,
        scratch_types=[
            pltpu.VMEM((CH,), jnp.int32),
            pltpu.VMEM((CH,), jnp.int32),
            pltpu.VMEM((CH, 128), jnp.float32),
            pltpu.VMEM((CH, 128), jnp.float32),
            pltpu.VMEM((_RSTR, 128), jnp.float32),
            pltpu.VMEM_SHARED((_RACC, 128), jnp.float32),
            pltpu.SemaphoreType.DMA,
            pltpu.SemaphoreType.DMA,
        ],
    )
    def k(*refs):
        vals_refs = refs[:nsub]
        idx_refs = refs[nsub:2 * nsub]
        out_refs = refs[2 * nsub:3 * nsub]
        idx0, idx1, vals0, vals1, zeros_v, acc, s0, s1 = refs[3 * nsub:]
        idx_b = (idx0, idx1)
        vals_b = (vals0, vals1)
        ssem = (s0, s1)
        c = lax.axis_index("c")
        s = lax.axis_index("s")
        zero16 = jnp.zeros((16,), jnp.float32)

        def zb(j, carry):
            zeros_v[j // 8, pl.ds((j % 8) * 16, 16)] = zero16
            return carry

        lax.fori_loop(0, _RSTR * 8, zb, 0)

        for sub in range(nsub):
            vals_hbm = vals_refs[sub]
            idx_hbm = idx_refs[sub]
            out_hbm = out_refs[sub]
            for p in range(_NPASS):
                r = _NPASS * c + p
                lo = r * _RQ
                pltpu.sync_copy(zeros_v, acc.at[pl.ds(s * _RSTR, _RSTR)])
                plsc.subcore_barrier()

                def stage(cc, b):
                    base = (cc * 16 + s) * CH
                    pltpu.sync_copy(idx_hbm.at[pl.ds(base, CH)], idx_b[b])
                    pltpu.sync_copy(vals_hbm.at[pl.ds(base, CH)], vals_b[b])

                    def remap(j, carry2):
                        v = idx_b[b][pl.ds(j * 16, 16)] - lo
                        ok = (v >= 0) & (v < _RQ)
                        idx_b[b][pl.ds(j * 16, 16)] = jnp.where(ok, v, _RQ)
                        return carry2

                    lax.fori_loop(0, CH // 16, remap, 0)
                    pltpu.async_copy(vals_b[b], acc.at[idx_b[b]], ssem[b],
                                     add=True)

                def wait_scatter(b):
                    pltpu.make_async_copy(vals_b[b], acc.at[idx_b[b]],
                                          ssem[b]).wait()

                for b in range(2):
                    stage(b, b)

                def body(i, carry):
                    for b in range(2):
                        wait_scatter(b)
                        stage(2 + 2 * i + b, b)
                    return carry

                lax.fori_loop(0, (per_sc - 2) // 2, body, 0)
                for b in range(2):
                    wait_scatter(b)
                plsc.subcore_barrier()
                pltpu.sync_copy(acc.at[pl.ds(s * _RSTR, _RSTR)],
                                out_hbm.at[r, pl.ds(s * _RSTR, _RSTR)])
                plsc.subcore_barrier()

    return k


_NP16 = 10112


@functools.lru_cache(maxsize=None)
def _sc_scalar_scatter_kernel(nsub, rows, CH):
    per_w = rows // CH // _NW
    mesh = plsc.VectorSubcoreMesh(core_axis_name="c", subcore_axis_name="s")
    outs = tuple(jax.ShapeDtypeStruct((_NW, _NP16), jnp.float32)
                 for _ in range(nsub))

    @functools.partial(
        pl.kernel,
        out_type=outs,
        mesh=mesh,
        scratch_types=[
            pltpu.VMEM((CH,), jnp.int32),
            pltpu.VMEM((CH,), jnp.float32),
            pltpu.VMEM((_NP16,), jnp.float32),
        ],
    )
    def k(*refs):
        vals_refs = refs[:nsub]
        idx_refs = refs[nsub:2 * nsub]
        out_refs = refs[2 * nsub:3 * nsub]
        idx_v, val_v, acc = refs[3 * nsub:]
        wid = lax.axis_index("s") * 2 + lax.axis_index("c")
        zero16 = jnp.zeros((16,), jnp.float32)

        for sub in range(nsub):
            vals_hbm = vals_refs[sub]
            idx_hbm = idx_refs[sub]
            out_hbm = out_refs[sub]

            def zb(j, carry):
                acc[pl.ds(j * 16, 16)] = zero16
                return carry

            lax.fori_loop(0, _NP16 // 16, zb, 0)

            def body(i, carry):
                base = (i * _NW + wid) * CH
                pltpu.sync_copy(idx_hbm.at[pl.ds(base, CH)], idx_v)
                pltpu.sync_copy(vals_hbm.at[pl.ds(base, CH)], val_v)

                def inner(j, carry2):
                    iv = idx_v[pl.ds(j * 16, 16)]
                    vv = val_v[pl.ds(j * 16, 16)]
                    plsc.addupdate_scatter(acc, [iv], vv)
                    return carry2

                lax.fori_loop(0, CH // 16, inner, 0)
                return carry

            lax.fori_loop(0, per_w, body, 0)
            pltpu.sync_copy(acc, out_hbm.at[wid])

    return k


def _sc_scalar_scatter(vals_list, idx_list):
    nsub = len(vals_list)
    rows = vals_list[0].shape[0]
    outs = _sc_scalar_scatter_kernel(nsub, rows, 2000)(*vals_list, *idx_list)
    if not isinstance(outs, (tuple, list)):
        outs = (outs,)
    return [jnp.sum(o, axis=0)[:_N] for o in outs]


def _sc_scatter128(vals_list, idx_list):
    nsub = len(vals_list)
    rows = vals_list[0].shape[0]
    outs = _sc_scatter128_kernel(nsub, rows, 400)(*vals_list, *idx_list)
    if not isinstance(outs, (tuple, list)):
        outs = (outs,)
    return [jnp.concatenate([o[r, :_RQ] for r in range(2 * _NPASS)],
                            axis=0)[:_N] for o in outs]


@functools.lru_cache(maxsize=None)
def _sc_scatter_add_kernel(rows, W, CH):
    per_sc = rows // CH // 16
    stripe = _NACC // 16
    mesh = plsc.VectorSubcoreMesh(core_axis_name="c", subcore_axis_name="s")

    @functools.partial(
        pl.kernel,
        out_type=jax.ShapeDtypeStruct((2, _NACC, W), jnp.float32),
        mesh=mesh,
        scratch_types=[
            pltpu.VMEM((CH,), jnp.int32),
            pltpu.VMEM((CH, W), jnp.float32),
            pltpu.VMEM((_ZR, W), jnp.float32),
            pltpu.VMEM_SHARED((_NACC, W), jnp.float32),
        ],
    )
    def k(vals_hbm, idx_hbm, out_hbm, idx_v, vals_v, zeros_v, acc):
        c = lax.axis_index("c")
        s = lax.axis_index("s")
        zero16 = jnp.zeros((16,), jnp.float32)
        wcol = W // 16
        lo = c * _NHALF

        def zb(j, carry):
            zeros_v[j // wcol, pl.ds((j % wcol) * 16, 16)] = zero16
            return carry

        lax.fori_loop(0, _ZR * wcol, zb, 0)
        pltpu.sync_copy(zeros_v.at[pl.ds(0, stripe)], acc.at[pl.ds(s * stripe, stripe)])
        plsc.subcore_barrier()

        def body(i, carry):
            base = (i * 16 + s) * CH
            pltpu.sync_copy(idx_hbm.at[pl.ds(base, CH)], idx_v)
            pltpu.sync_copy(vals_hbm.at[pl.ds(base, CH)], vals_v)

            def remap(j, carry2):
                v = idx_v[pl.ds(j * 16, 16)] - lo
                ok = (v >= 0) & (v < _NHALF)
                idx_v[pl.ds(j * 16, 16)] = jnp.where(ok, v, _NHALF)
                return carry2

            lax.fori_loop(0, CH // 16, remap, 0)
            pltpu.sync_copy(vals_v, acc.at[idx_v], add=True)
            return carry

        lax.fori_loop(0, per_sc, body, 0)
        plsc.subcore_barrier()
        pltpu.sync_copy(acc.at[pl.ds(s * stripe, stripe)],
                        out_hbm.at[c, pl.ds(s * stripe, stripe)])

    return k


def _sc_scatter_add(vals, idx, CH):
    return _sc_scatter_add_kernel(vals.shape[0], vals.shape[1], CH)(vals, idx)


def _assemble_halves(p):
    return jnp.concatenate([p[0, :_NHALF], p[1, :_N - _NHALF]], axis=0)


def _silu(x):
    return x * jax.nn.sigmoid(x)


def _dsilu(x):
    s = jax.nn.sigmoid(x)
    return s * (1.0 + x * (1.0 - s))


def _gvec_scale_body(glen_ref, lraw_ref, out_ref):
    glen = glen_ref[...]
    lraw = lraw_ref[...]
    mask = (lraw < _R_MAX).astype(jnp.float32)
    out_ref[...] = glen * mask / lraw


def _gvec_scale(glen, len_raw):
    g2 = glen.reshape(_E // 128, 128)
    l2 = len_raw.reshape(_E // 128, 128)
    out = pl.pallas_call(
        _gvec_scale_body,
        out_shape=jax.ShapeDtypeStruct(g2.shape, jnp.float32),
    )(g2, l2)
    return out.reshape(_E)


def kernel(z, pos, edge_index, emb, W_rad, W_ace, b_ace, W_mp, Wq, Wk, Wv, Wo,
           dw, db, ln_g, ln_b, Wr1, br1, Wr2, br2, gw, gb,
           afW1, afb1, afW2, afb2, asW1, asb1, asW2, asb2):
    src = edge_index[0]
    dst = edge_index[1]

    edge_vec = pos[src] - pos[dst]
    d2 = jnp.sum(edge_vec ** 2, axis=1) + 1e-12
    len_raw = jnp.sqrt(d2)
    edge_len = jnp.minimum(len_raw, _R_MAX)
    u = edge_len / _R_MAX
    dl = edge_len + 1e-9

    n = jnp.arange(1, _R + 1, dtype=jnp.float32)
    c = jnp.sqrt(2.0 / _R_MAX)
    sin_t = jnp.sin(n[None, :] * jnp.pi * u[:, None])
    cos_t = jnp.cos(n[None, :] * jnp.pi * u[:, None])
    a1, a2, a3 = 21.0, 35.0, 15.0
    u4 = u ** 4
    u5 = u4 * u
    env = 1.0 - a1 * u5 + a2 * u5 * u - a3 * u5 * u * u
    denv = (-5 * a1) * u4 + (6 * a2) * u5 + (-7 * a3) * u5 * u
    rb0 = c * sin_t / dl[:, None]
    rb = rb0 * env[:, None]
    drb = (c * env[:, None]) * ((n[None, :] * jnp.pi / _R_MAX) * cos_t / dl[:, None]
                                - sin_t / (dl * dl)[:, None]) \
        + rb0 * (denv / _R_MAX)[:, None]

    w = jnp.exp(-_SHARP * u)
    zp = jnp.zeros((10240,), jnp.int32).at[:_N].set(z)
    h0p = _sc_gather(emb, zp, 320)
    h0 = h0p[:_N]

    rw = rb @ W_rad
    h0s = _sc_gather(h0p, src, 200)
    agg, = _sc_scatter128([h0s * rw], [dst])
    pre1 = agg @ W_ace + b_ace
    h1 = _silu(pre1)

    h1s = _sc_gather(h1, src, 200)
    m = jax.ops.segment_sum(h1s * w[:, None], dst, num_segments=_N)
    deg = jax.ops.segment_sum(w, dst, num_segments=_N) + 1e-9
    md = m / deg[:, None]
    h2 = h1 + md @ W_mp

    q = h2 @ Wq
    k = h2 @ Wk
    v = h2 @ Wv
    ks = _sc_gather(k, src, 200)
    qd = _sc_gather(q, dst, 200)
    vs = _sc_gather(v, src, 200)
    inv_sqrt_h = 1.0 / jnp.sqrt(float(_H))
    qk = jnp.sum(qd * ks, axis=1) * inv_sqrt_h
    sig_arg = dw * edge_len + db
    decay = jax.nn.softplus(sig_arg)
    scores = qk - decay * edge_len
    a = jnp.exp(scores)
    denom = jax.ops.segment_sum(a, dst, num_segments=_N) + 1e-9
    num = jax.ops.segment_sum(a[:, None] * vs, dst, num_segments=_N)
    attn = num / denom[:, None]
    h3 = h2 + attn @ Wo

    mu = jnp.mean(h3, axis=1, keepdims=True)
    var = jnp.var(h3, axis=1, keepdims=True)
    std = jnp.sqrt(var + 1e-5)
    xhat = (h3 - mu) / std
    scalars = xhat * ln_g + ln_b

    t1 = scalars @ Wr1 + br1
    Enode = _silu(t1) @ Wr2 + br2
    Etot = jnp.sum(Enode)

    Gt1 = _dsilu(t1) * Wr2[:, 0][None, :]
    Gscal = Gt1 @ Wr1.T
    Gxhat = Gscal * ln_g[None, :]
    Gh3 = (Gxhat - jnp.mean(Gxhat, axis=1, keepdims=True)
           - xhat * jnp.mean(Gxhat * xhat, axis=1, keepdims=True)) / std
    Gattn = Gh3 @ Wo.T
    P = Gattn / denom[:, None]
    beta = jnp.sum(P * attn, axis=1)
    beta_d = beta[dst]
    Pd = _sc_gather(P, dst, 200)
    ds = a * (jnp.sum(Pd * vs, axis=1) - beta_d)
    Gv, Gq, Gk = _sc_scatter128(
        [a[:, None] * Pd, (ds * inv_sqrt_h)[:, None] * ks,
         (ds * inv_sqrt_h)[:, None] * qd],
        [src, dst, src])
    sig = jax.nn.sigmoid(sig_arg)
    glen = -ds * (decay + sig * dw * edge_len)
    G2 = Gh3 + Gq @ Wq.T + Gk @ Wk.T + Gv @ Wv.T
    Gmd = G2 @ W_mp.T
    Q = Gmd / deg[:, None]
    gamma = -jnp.sum(Q * md, axis=1)
    gamma_d = gamma[dst]
    Qd = _sc_gather(Q, dst, 200)
    gh1s, = _sc_scatter128([w[:, None] * Qd], [src])
    Gh1 = G2 + gh1s
    gw_e = jnp.sum(Qd * h1s, axis=1) + gamma_d
    glen = glen + gw_e * (-_SHARP / _R_MAX) * w
    Gpre1 = Gh1 * _dsilu(pre1)
    Gagg = Gpre1 @ W_ace.T
    D = _sc_gather(Gagg, dst, 200) * h0s
    grb = D @ W_rad.T
    glen = glen + jnp.sum(grb * drb, axis=1)
    gscale = _gvec_scale(glen, len_raw)
    gvec = gscale[:, None] * edge_vec
    gpos = jax.ops.segment_sum(gvec, src, num_segments=_N) \
        - jax.ops.segment_sum(gvec, dst, num_segments=_N)
    F = -gpos

    mean_edge = jnp.mean(edge_len)
    gate = jax.nn.sigmoid(mean_edge * gw[0, 0] + gb[0])
    aux_force = gate * (_silu(scalars @ afW1 + afb1) @ afW2 + afb2)
    pooled = jnp.mean(scalars, axis=0, keepdims=True)
    stress_voigt = (_silu(pooled @ asW1 + asb1) @ asW2 + asb2).reshape(-1)
    S = jnp.zeros((3, 3), dtype=jnp.float32)
    return (Etot, F, S, aux_force, stress_voigt)


# speedup vs baseline: 1.2809x; 1.2809x over previous
import functools

import jax
import jax.numpy as jnp
from jax import lax
from jax.experimental import pallas as pl
from jax.experimental.pallas import tpu as pltpu
from jax.experimental.pallas import tpu_sc as plsc

_N = 10000
_E = 320000
_H = 128
_R = 8
_R_MAX = 5.0
_SHARP = 6.0


_NW = 32
_NHALF = 5056
_NACC = 5120
_ZR = 320


@functools.lru_cache(maxsize=None)
def _sc_gather_kernel(T, W, rows, CH):
    per_w = rows // CH // _NW
    mesh = plsc.VectorSubcoreMesh(core_axis_name="c", subcore_axis_name="s")

    @functools.partial(
        pl.kernel,
        out_type=jax.ShapeDtypeStruct((rows, W), jnp.float32),
        mesh=mesh,
        scratch_types=[
            pltpu.VMEM((CH,), jnp.int32),
            pltpu.VMEM((CH,), jnp.int32),
            pltpu.VMEM((CH, W), jnp.float32),
            pltpu.VMEM((CH, W), jnp.float32),
            pltpu.SemaphoreType.DMA,
            pltpu.SemaphoreType.DMA,
            pltpu.SemaphoreType.DMA,
            pltpu.SemaphoreType.DMA,
        ],
    )
    def k(table_hbm, idx_hbm, out_hbm, idx0, idx1, rows0, rows1,
          g0, g1, w0, w1):
        wid = lax.axis_index("s") * 2 + lax.axis_index("c")
        idx_b = (idx0, idx1)
        rows_b = (rows0, rows1)
        gsem = (g0, g1)
        wsem = (w0, w1)

        def base_of(c):
            return (c * _NW + wid) * CH

        if per_w == 1:
            pltpu.sync_copy(idx_hbm.at[pl.ds(base_of(0), CH)], idx0)
            pltpu.async_copy(table_hbm.at[idx0], rows0, g0).wait()
            pltpu.sync_copy(rows0, out_hbm.at[pl.ds(base_of(0), CH)])
            return

        for b in range(2):
            pltpu.sync_copy(idx_hbm.at[pl.ds(base_of(b), CH)], idx_b[b])
            pltpu.async_copy(table_hbm.at[idx_b[b]], rows_b[b], gsem[b])

        def wait_gather(b):
            pltpu.make_async_copy(
                table_hbm.at[idx_b[b]], rows_b[b], gsem[b]).wait()

        def wait_write(b):
            pltpu.make_async_copy(
                rows_b[b], out_hbm.at[pl.ds(0, CH)], wsem[b]).wait()

        def body(i, carry):
            for b in range(2):
                cc = 2 + 2 * i + b
                wait_gather(b)
                pltpu.async_copy(
                    rows_b[b], out_hbm.at[pl.ds(base_of(cc - 2), CH)], wsem[b])
                pltpu.sync_copy(idx_hbm.at[pl.ds(base_of(cc), CH)], idx_b[b])
                wait_write(b)
                pltpu.async_copy(table_hbm.at[idx_b[b]], rows_b[b], gsem[b])
            return carry

        lax.fori_loop(0, (per_w - 2) // 2, body, 0)
        for b in range(2):
            cc = per_w - 2 + b
            wait_gather(b)
            pltpu.sync_copy(rows_b[b], out_hbm.at[pl.ds(base_of(cc), CH)])

    return k


def _sc_gather(table, idx, CH):
    return _sc_gather_kernel(table.shape[0], table.shape[1], idx.shape[0],
                             CH)(table, idx)


_RQ = 2528
_RACC = 2560
_RSTR = 160
_NPASS = 2


@functools.lru_cache(maxsize=None)
def _sc_scatter128_kernel(nsub, rows, CH):
    per_sc = rows // CH // 16
    mesh = plsc.VectorSubcoreMesh(core_axis_name="c", subcore_axis_name="s")
    outs = tuple(jax.ShapeDtypeStruct((2 * _NPASS, _RACC, 128), jnp.float32)
                 for _ in range(nsub))

    @functools.partial(
        pl.kernel,
        out_type=outs,
        mesh=mesh,
        scratch_types=[
            pltpu.VMEM((CH,), jnp.int32),
            pltpu.VMEM((CH, 128), jnp.float32),
            pltpu.VMEM((_RSTR, 128), jnp.float32),
            pltpu.VMEM_SHARED((_RACC, 128), jnp.float32),
        ],
    )
    def k(*refs):
        vals_refs = refs[:nsub]
        idx_refs = refs[nsub:2 * nsub]
        out_refs = refs[2 * nsub:3 * nsub]
        idx_v, vals_v, zeros_v, acc = refs[3 * nsub:]
        c = lax.axis_index("c")
        s = lax.axis_index("s")
        zero16 = jnp.zeros((16,), jnp.float32)

        def zb(j, carry):
            zeros_v[j // 8, pl.ds((j % 8) * 16, 16)] = zero16
            return carry

        lax.fori_loop(0, _RSTR * 8, zb, 0)

        for sub in range(nsub):
            vals_hbm = vals_refs[sub]
            idx_hbm = idx_refs[sub]
            out_hbm = out_refs[sub]
            for p in range(_NPASS):
                r = _NPASS * c + p
                lo = r * _RQ
                pltpu.sync_copy(zeros_v, acc.at[pl.ds(s * _RSTR, _RSTR)])
                plsc.subcore_barrier()

                def body(i, carry):
                    base = (i * 16 + s) * CH
                    pltpu.sync_copy(idx_hbm.at[pl.ds(base, CH)], idx_v)
                    pltpu.sync_copy(vals_hbm.at[pl.ds(base, CH)], vals_v)

                    def remap(j, carry2):
                        v = idx_v[pl.ds(j * 16, 16)] - lo
                        ok = (v >= 0) & (v < _RQ)
                        idx_v[pl.ds(j * 16, 16)] = jnp.where(ok, v, _RQ)
                        return carry2

                    lax.fori_loop(0, CH // 16, remap, 0)
                    pltpu.sync_copy(vals_v, acc.at[idx_v], add=True)
                    return carry

                lax.fori_loop(0, per_sc, body, 0)
                plsc.subcore_barrier()
                pltpu.sync_copy(acc.at[pl.ds(s * _RSTR, _RSTR)],
                                out_hbm.at[r, pl.ds(s * _RSTR, _RSTR)])
                plsc.subcore_barrier()

    return k


_NP16 = 10112


@functools.lru_cache(maxsize=None)
def _sc_scalar_scatter_kernel(nsub, rows, CH):
    per_w = rows // CH // _NW
    mesh = plsc.VectorSubcoreMesh(core_axis_name="c", subcore_axis_name="s")
    outs = tuple(jax.ShapeDtypeStruct((_NW, _NP16), jnp.float32)
                 for _ in range(nsub))

    @functools.partial(
        pl.kernel,
        out_type=outs,
        mesh=mesh,
        scratch_types=[
            pltpu.VMEM((CH,), jnp.int32),
            pltpu.VMEM((CH,), jnp.float32),
            pltpu.VMEM((_NP16,), jnp.float32),
        ],
    )
    def k(*refs):
        vals_refs = refs[:nsub]
        idx_refs = refs[nsub:2 * nsub]
        out_refs = refs[2 * nsub:3 * nsub]
        idx_v, val_v, acc = refs[3 * nsub:]
        wid = lax.axis_index("s") * 2 + lax.axis_index("c")
        zero16 = jnp.zeros((16,), jnp.float32)

        for sub in range(nsub):
            vals_hbm = vals_refs[sub]
            idx_hbm = idx_refs[sub]
            out_hbm = out_refs[sub]

            def zb(j, carry):
                acc[pl.ds(j * 16, 16)] = zero16
                return carry

            lax.fori_loop(0, _NP16 // 16, zb, 0)

            def body(i, carry):
                base = (i * _NW + wid) * CH
                pltpu.sync_copy(idx_hbm.at[pl.ds(base, CH)], idx_v)
                pltpu.sync_copy(vals_hbm.at[pl.ds(base, CH)], val_v)

                def inner(j, carry2):
                    iv = idx_v[pl.ds(j * 16, 16)]
                    vv = val_v[pl.ds(j * 16, 16)]
                    plsc.addupdate_scatter(acc, [iv], vv)
                    return carry2

                lax.fori_loop(0, CH // 16, inner, 0)
                return carry

            lax.fori_loop(0, per_w, body, 0)
            pltpu.sync_copy(acc, out_hbm.at[wid])

    return k


def _sc_scalar_scatter(vals_list, idx_list):
    nsub = len(vals_list)
    rows = vals_list[0].shape[0]
    outs = _sc_scalar_scatter_kernel(nsub, rows, 2000)(*vals_list, *idx_list)
    if not isinstance(outs, (tuple, list)):
        outs = (outs,)
    return [jnp.sum(o, axis=0)[:_N] for o in outs]


def _sc_scatter128(vals_list, idx_list):
    nsub = len(vals_list)
    rows = vals_list[0].shape[0]
    outs = _sc_scatter128_kernel(nsub, rows, 400)(*vals_list, *idx_list)
    if not isinstance(outs, (tuple, list)):
        outs = (outs,)
    return [jnp.concatenate([o[r, :_RQ] for r in range(2 * _NPASS)],
                            axis=0)[:_N] for o in outs]


@functools.lru_cache(maxsize=None)
def _sc_scatter_add_kernel(rows, W, CH):
    per_sc = rows // CH // 16
    stripe = _NACC // 16
    mesh = plsc.VectorSubcoreMesh(core_axis_name="c", subcore_axis_name="s")

    @functools.partial(
        pl.kernel,
        out_type=jax.ShapeDtypeStruct((2, _NACC, W), jnp.float32),
        mesh=mesh,
        scratch_types=[
            pltpu.VMEM((CH,), jnp.int32),
            pltpu.VMEM((CH, W), jnp.float32),
            pltpu.VMEM((_ZR, W), jnp.float32),
            pltpu.VMEM_SHARED((_NACC, W), jnp.float32),
        ],
    )
    def k(vals_hbm, idx_hbm, out_hbm, idx_v, vals_v, zeros_v, acc):
        c = lax.axis_index("c")
        s = lax.axis_index("s")
        zero16 = jnp.zeros((16,), jnp.float32)
        wcol = W // 16
        lo = c * _NHALF

        def zb(j, carry):
            zeros_v[j // wcol, pl.ds((j % wcol) * 16, 16)] = zero16
            return carry

        lax.fori_loop(0, _ZR * wcol, zb, 0)
        pltpu.sync_copy(zeros_v.at[pl.ds(0, stripe)], acc.at[pl.ds(s * stripe, stripe)])
        plsc.subcore_barrier()

        def body(i, carry):
            base = (i * 16 + s) * CH
            pltpu.sync_copy(idx_hbm.at[pl.ds(base, CH)], idx_v)
            pltpu.sync_copy(vals_hbm.at[pl.ds(base, CH)], vals_v)

            def remap(j, carry2):
                v = idx_v[pl.ds(j * 16, 16)] - lo
                ok = (v >= 0) & (v < _NHALF)
                idx_v[pl.ds(j * 16, 16)] = jnp.where(ok, v, _NHALF)
                return carry2

            lax.fori_loop(0, CH // 16, remap, 0)
            pltpu.sync_copy(vals_v, acc.at[idx_v], add=True)
            return carry

        lax.fori_loop(0, per_sc, body, 0)
        plsc.subcore_barrier()
        pltpu.sync_copy(acc.at[pl.ds(s * stripe, stripe)],
                        out_hbm.at[c, pl.ds(s * stripe, stripe)])

    return k


def _sc_scatter_add(vals, idx, CH):
    return _sc_scatter_add_kernel(vals.shape[0], vals.shape[1], CH)(vals, idx)


def _assemble_halves(p):
    return jnp.concatenate([p[0, :_NHALF], p[1, :_N - _NHALF]], axis=0)


def _silu(x):
    return x * jax.nn.sigmoid(x)


def _dsilu(x):
    s = jax.nn.sigmoid(x)
    return s * (1.0 + x * (1.0 - s))


def _gvec_scale_body(glen_ref, lraw_ref, out_ref):
    glen = glen_ref[...]
    lraw = lraw_ref[...]
    mask = (lraw < _R_MAX).astype(jnp.float32)
    out_ref[...] = glen * mask / lraw


def _gvec_scale(glen, len_raw):
    g2 = glen.reshape(_E // 128, 128)
    l2 = len_raw.reshape(_E // 128, 128)
    out = pl.pallas_call(
        _gvec_scale_body,
        out_shape=jax.ShapeDtypeStruct(g2.shape, jnp.float32),
    )(g2, l2)
    return out.reshape(_E)


def kernel(z, pos, edge_index, emb, W_rad, W_ace, b_ace, W_mp, Wq, Wk, Wv, Wo,
           dw, db, ln_g, ln_b, Wr1, br1, Wr2, br2, gw, gb,
           afW1, afb1, afW2, afb2, asW1, asb1, asW2, asb2):
    src = edge_index[0]
    dst = edge_index[1]

    edge_vec = pos[src] - pos[dst]
    d2 = jnp.sum(edge_vec ** 2, axis=1) + 1e-12
    len_raw = jnp.sqrt(d2)
    edge_len = jnp.minimum(len_raw, _R_MAX)
    u = edge_len / _R_MAX
    dl = edge_len + 1e-9

    n = jnp.arange(1, _R + 1, dtype=jnp.float32)
    c = jnp.sqrt(2.0 / _R_MAX)
    sin_t = jnp.sin(n[None, :] * jnp.pi * u[:, None])
    cos_t = jnp.cos(n[None, :] * jnp.pi * u[:, None])
    a1, a2, a3 = 21.0, 35.0, 15.0
    u4 = u ** 4
    u5 = u4 * u
    env = 1.0 - a1 * u5 + a2 * u5 * u - a3 * u5 * u * u
    denv = (-5 * a1) * u4 + (6 * a2) * u5 + (-7 * a3) * u5 * u
    rb0 = c * sin_t / dl[:, None]
    rb = rb0 * env[:, None]
    drb = (c * env[:, None]) * ((n[None, :] * jnp.pi / _R_MAX) * cos_t / dl[:, None]
                                - sin_t / (dl * dl)[:, None]) \
        + rb0 * (denv / _R_MAX)[:, None]

    w = jnp.exp(-_SHARP * u)
    zp = jnp.zeros((10240,), jnp.int32).at[:_N].set(z)
    h0p = _sc_gather(emb, zp, 320)
    h0 = h0p[:_N]

    rw = rb @ W_rad
    h0s = _sc_gather(h0p, src, 200)
    agg, = _sc_scatter128([h0s * rw], [dst])
    pre1 = agg @ W_ace + b_ace
    h1 = _silu(pre1)

    h1s = _sc_gather(h1, src, 200)
    m, = _sc_scatter128([h1s * w[:, None]], [dst])
    deg = jax.ops.segment_sum(w, dst, num_segments=_N) + 1e-9
    md = m / deg[:, None]
    h2 = h1 + md @ W_mp

    q = h2 @ Wq
    k = h2 @ Wk
    v = h2 @ Wv
    ks = _sc_gather(k, src, 200)
    qd = _sc_gather(q, dst, 200)
    vs = _sc_gather(v, src, 200)
    inv_sqrt_h = 1.0 / jnp.sqrt(float(_H))
    qk = jnp.sum(qd * ks, axis=1) * inv_sqrt_h
    sig_arg = dw * edge_len + db
    decay = jax.nn.softplus(sig_arg)
    scores = qk - decay * edge_len
    a = jnp.exp(scores)
    denom = jax.ops.segment_sum(a, dst, num_segments=_N) + 1e-9
    num, = _sc_scatter128([a[:, None] * vs], [dst])
    attn = num / denom[:, None]
    h3 = h2 + attn @ Wo

    mu = jnp.mean(h3, axis=1, keepdims=True)
    var = jnp.var(h3, axis=1, keepdims=True)
    std = jnp.sqrt(var + 1e-5)
    xhat = (h3 - mu) / std
    scalars = xhat * ln_g + ln_b

    t1 = scalars @ Wr1 + br1
    Enode = _silu(t1) @ Wr2 + br2
    Etot = jnp.sum(Enode)

    Gt1 = _dsilu(t1) * Wr2[:, 0][None, :]
    Gscal = Gt1 @ Wr1.T
    Gxhat = Gscal * ln_g[None, :]
    Gh3 = (Gxhat - jnp.mean(Gxhat, axis=1, keepdims=True)
           - xhat * jnp.mean(Gxhat * xhat, axis=1, keepdims=True)) / std
    Gattn = Gh3 @ Wo.T
    P = Gattn / denom[:, None]
    beta = jnp.sum(P * attn, axis=1)
    beta_d = beta[dst]
    Pd = _sc_gather(P, dst, 200)
    ds = a * (jnp.sum(Pd * vs, axis=1) - beta_d)
    Gv, Gq, Gk = _sc_scatter128(
        [a[:, None] * Pd, (ds * inv_sqrt_h)[:, None] * ks,
         (ds * inv_sqrt_h)[:, None] * qd],
        [src, dst, src])
    sig = jax.nn.sigmoid(sig_arg)
    glen = -ds * (decay + sig * dw * edge_len)
    G2 = Gh3 + Gq @ Wq.T + Gk @ Wk.T + Gv @ Wv.T
    Gmd = G2 @ W_mp.T
    Q = Gmd / deg[:, None]
    gamma = -jnp.sum(Q * md, axis=1)
    gamma_d = gamma[dst]
    Qd = _sc_gather(Q, dst, 200)
    gh1s, = _sc_scatter128([w[:, None] * Qd], [src])
    Gh1 = G2 + gh1s
    gw_e = jnp.sum(Qd * h1s, axis=1) + gamma_d
    glen = glen + gw_e * (-_SHARP / _R_MAX) * w
    Gpre1 = Gh1 * _dsilu(pre1)
    Gagg = Gpre1 @ W_ace.T
    D = _sc_gather(Gagg, dst, 200) * h0s
    grb = D @ W_rad.T
    glen = glen + jnp.sum(grb * drb, axis=1)
    gscale = _gvec_scale(glen, len_raw)
    gvec = gscale[:, None] * edge_vec
    gpos = jax.ops.segment_sum(gvec, src, num_segments=_N) \
        - jax.ops.segment_sum(gvec, dst, num_segments=_N)
    F = -gpos

    mean_edge = jnp.mean(edge_len)
    gate = jax.nn.sigmoid(mean_edge * gw[0, 0] + gb[0])
    aux_force = gate * (_silu(scalars @ afW1 + afb1) @ afW2 + afb2)
    pooled = jnp.mean(scalars, axis=0, keepdims=True)
    stress_voigt = (_silu(pooled @ asW1 + asb1) @ asW2 + asb2).reshape(-1)
    S = jnp.zeros((3, 3), dtype=jnp.float32)
    return (Etot, F, S, aux_force, stress_voigt)


# speedup vs baseline: 1.2813x; 1.0003x over previous
import functools

import jax
import jax.numpy as jnp
from jax import lax
from jax.experimental import pallas as pl
from jax.experimental.pallas import tpu as pltpu
from jax.experimental.pallas import tpu_sc as plsc

_N = 10000
_E = 320000
_H = 128
_R = 8
_R_MAX = 5.0
_SHARP = 6.0


_NW = 32


@functools.lru_cache(maxsize=None)
def _sc_gather_kernel(T, W, rows, CH):
    per_w = rows // CH // _NW
    mesh = plsc.VectorSubcoreMesh(core_axis_name="c", subcore_axis_name="s")

    @functools.partial(
        pl.kernel,
        out_type=jax.ShapeDtypeStruct((rows, W), jnp.float32),
        mesh=mesh,
        scratch_types=[
            pltpu.VMEM((CH,), jnp.int32),
            pltpu.VMEM((CH,), jnp.int32),
            pltpu.VMEM((CH, W), jnp.float32),
            pltpu.VMEM((CH, W), jnp.float32),
            pltpu.SemaphoreType.DMA,
            pltpu.SemaphoreType.DMA,
            pltpu.SemaphoreType.DMA,
            pltpu.SemaphoreType.DMA,
        ],
    )
    def k(table_hbm, idx_hbm, out_hbm, idx0, idx1, rows0, rows1,
          g0, g1, w0, w1):
        wid = lax.axis_index("s") * 2 + lax.axis_index("c")
        idx_b = (idx0, idx1)
        rows_b = (rows0, rows1)
        gsem = (g0, g1)
        wsem = (w0, w1)

        def base_of(c):
            return (c * _NW + wid) * CH

        if per_w == 1:
            pltpu.sync_copy(idx_hbm.at[pl.ds(base_of(0), CH)], idx0)
            pltpu.async_copy(table_hbm.at[idx0], rows0, g0).wait()
            pltpu.sync_copy(rows0, out_hbm.at[pl.ds(base_of(0), CH)])
            return

        for b in range(2):
            pltpu.sync_copy(idx_hbm.at[pl.ds(base_of(b), CH)], idx_b[b])
            pltpu.async_copy(table_hbm.at[idx_b[b]], rows_b[b], gsem[b])

        def wait_gather(b):
            pltpu.make_async_copy(
                table_hbm.at[idx_b[b]], rows_b[b], gsem[b]).wait()

        def wait_write(b):
            pltpu.make_async_copy(
                rows_b[b], out_hbm.at[pl.ds(0, CH)], wsem[b]).wait()

        def body(i, carry):
            for b in range(2):
                cc = 2 + 2 * i + b
                wait_gather(b)
                pltpu.async_copy(
                    rows_b[b], out_hbm.at[pl.ds(base_of(cc - 2), CH)], wsem[b])
                pltpu.sync_copy(idx_hbm.at[pl.ds(base_of(cc), CH)], idx_b[b])
                wait_write(b)
                pltpu.async_copy(table_hbm.at[idx_b[b]], rows_b[b], gsem[b])
            return carry

        lax.fori_loop(0, (per_w - 2) // 2, body, 0)
        for b in range(2):
            cc = per_w - 2 + b
            wait_gather(b)
            pltpu.sync_copy(rows_b[b], out_hbm.at[pl.ds(base_of(cc), CH)])

    return k


def _sc_gather(table, idx, CH):
    return _sc_gather_kernel(table.shape[0], table.shape[1], idx.shape[0],
                             CH)(table, idx)


_RQ = 2528
_RACC = 2560
_RSTR = 160
_NPASS = 2


@functools.lru_cache(maxsize=None)
def _sc_scatter128_kernel(nsub, rows, CH):
    per_sc = rows // CH // 16
    mesh = plsc.VectorSubcoreMesh(core_axis_name="c", subcore_axis_name="s")
    outs = tuple(jax.ShapeDtypeStruct((2 * _NPASS, _RACC, 128), jnp.float32)
                 for _ in range(nsub))

    @functools.partial(
        pl.kernel,
        out_type=outs,
        mesh=mesh,
        scratch_types=[
            pltpu.VMEM((CH,), jnp.int32),
            pltpu.VMEM((CH, 128), jnp.float32),
            pltpu.VMEM((_RSTR, 128), jnp.float32),
            pltpu.VMEM_SHARED((_RACC, 128), jnp.float32),
        ],
    )
    def k(*refs):
        vals_refs = refs[:nsub]
        idx_refs = refs[nsub:2 * nsub]
        out_refs = refs[2 * nsub:3 * nsub]
        idx_v, vals_v, zeros_v, acc = refs[3 * nsub:]
        c = lax.axis_index("c")
        s = lax.axis_index("s")
        zero16 = jnp.zeros((16,), jnp.float32)

        def zb(j, carry):
            zeros_v[j // 8, pl.ds((j % 8) * 16, 16)] = zero16
            return carry

        lax.fori_loop(0, _RSTR * 8, zb, 0)

        for sub in range(nsub):
            vals_hbm = vals_refs[sub]
            idx_hbm = idx_refs[sub]
            out_hbm = out_refs[sub]
            for p in range(_NPASS):
                r = _NPASS * c + p
                lo = r * _RQ
                pltpu.sync_copy(zeros_v, acc.at[pl.ds(s * _RSTR, _RSTR)])
                plsc.subcore_barrier()

                def body(i, carry):
                    base = (i * 16 + s) * CH
                    pltpu.sync_copy(idx_hbm.at[pl.ds(base, CH)], idx_v)
                    pltpu.sync_copy(vals_hbm.at[pl.ds(base, CH)], vals_v)

                    def remap(j, carry2):
                        v = idx_v[pl.ds(j * 16, 16)] - lo
                        ok = (v >= 0) & (v < _RQ)
                        idx_v[pl.ds(j * 16, 16)] = jnp.where(ok, v, _RQ)
                        return carry2

                    lax.fori_loop(0, CH // 16, remap, 0)
                    pltpu.sync_copy(vals_v, acc.at[idx_v], add=True)
                    return carry

                lax.fori_loop(0, per_sc, body, 0)
                plsc.subcore_barrier()
                pltpu.sync_copy(acc.at[pl.ds(s * _RSTR, _RSTR)],
                                out_hbm.at[r, pl.ds(s * _RSTR, _RSTR)])
                plsc.subcore_barrier()

    return k


def _sc_scatter128(vals_list, idx_list):
    nsub = len(vals_list)
    rows = vals_list[0].shape[0]
    outs = _sc_scatter128_kernel(nsub, rows, 400)(*vals_list, *idx_list)
    if not isinstance(outs, (tuple, list)):
        outs = (outs,)
    return [jnp.concatenate([o[r, :_RQ] for r in range(2 * _NPASS)],
                            axis=0)[:_N] for o in outs]


def _silu(x):
    return x * jax.nn.sigmoid(x)


def _dsilu(x):
    s = jax.nn.sigmoid(x)
    return s * (1.0 + x * (1.0 - s))


def _gvec_scale_body(glen_ref, lraw_ref, out_ref):
    glen = glen_ref[...]
    lraw = lraw_ref[...]
    mask = (lraw < _R_MAX).astype(jnp.float32)
    out_ref[...] = glen * mask / lraw


def _gvec_scale(glen, len_raw):
    g2 = glen.reshape(_E // 128, 128)
    l2 = len_raw.reshape(_E // 128, 128)
    out = pl.pallas_call(
        _gvec_scale_body,
        out_shape=jax.ShapeDtypeStruct(g2.shape, jnp.float32),
    )(g2, l2)
    return out.reshape(_E)


def kernel(z, pos, edge_index, emb, W_rad, W_ace, b_ace, W_mp, Wq, Wk, Wv, Wo,
           dw, db, ln_g, ln_b, Wr1, br1, Wr2, br2, gw, gb,
           afW1, afb1, afW2, afb2, asW1, asb1, asW2, asb2):
    src = edge_index[0]
    dst = edge_index[1]

    edge_vec = pos[src] - pos[dst]
    d2 = jnp.sum(edge_vec ** 2, axis=1) + 1e-12
    len_raw = jnp.sqrt(d2)
    edge_len = jnp.minimum(len_raw, _R_MAX)
    u = edge_len / _R_MAX
    dl = edge_len + 1e-9

    n = jnp.arange(1, _R + 1, dtype=jnp.float32)
    c = jnp.sqrt(2.0 / _R_MAX)
    sin_t = jnp.sin(n[None, :] * jnp.pi * u[:, None])
    cos_t = jnp.cos(n[None, :] * jnp.pi * u[:, None])
    a1, a2, a3 = 21.0, 35.0, 15.0
    u4 = u ** 4
    u5 = u4 * u
    env = 1.0 - a1 * u5 + a2 * u5 * u - a3 * u5 * u * u
    denv = (-5 * a1) * u4 + (6 * a2) * u5 + (-7 * a3) * u5 * u
    rb0 = c * sin_t / dl[:, None]
    rb = rb0 * env[:, None]
    drb = (c * env[:, None]) * ((n[None, :] * jnp.pi / _R_MAX) * cos_t / dl[:, None]
                                - sin_t / (dl * dl)[:, None]) \
        + rb0 * (denv / _R_MAX)[:, None]

    w = jnp.exp(-_SHARP * u)
    zp = jnp.zeros((10240,), jnp.int32).at[:_N].set(z)
    h0p = _sc_gather(emb, zp, 320)
    h0 = h0p[:_N]

    rw = rb @ W_rad
    h0s = _sc_gather(h0p, src, 200)
    agg, = _sc_scatter128([h0s * rw], [dst])
    pre1 = agg @ W_ace + b_ace
    h1 = _silu(pre1)

    h1s = _sc_gather(h1, src, 200)
    m, = _sc_scatter128([h1s * w[:, None]], [dst])
    deg = jax.ops.segment_sum(w, dst, num_segments=_N) + 1e-9
    md = m / deg[:, None]
    h2 = h1 + md @ W_mp

    q = h2 @ Wq
    k = h2 @ Wk
    v = h2 @ Wv
    ks = _sc_gather(k, src, 200)
    qd = _sc_gather(q, dst, 200)
    vs = _sc_gather(v, src, 200)
    inv_sqrt_h = 1.0 / jnp.sqrt(float(_H))
    qk = jnp.sum(qd * ks, axis=1) * inv_sqrt_h
    sig_arg = dw * edge_len + db
    decay = jax.nn.softplus(sig_arg)
    scores = qk - decay * edge_len
    a = jnp.exp(scores)
    denom = jax.ops.segment_sum(a, dst, num_segments=_N) + 1e-9
    num, = _sc_scatter128([a[:, None] * vs], [dst])
    attn = num / denom[:, None]
    h3 = h2 + attn @ Wo

    mu = jnp.mean(h3, axis=1, keepdims=True)
    var = jnp.var(h3, axis=1, keepdims=True)
    std = jnp.sqrt(var + 1e-5)
    xhat = (h3 - mu) / std
    scalars = xhat * ln_g + ln_b

    t1 = scalars @ Wr1 + br1
    Enode = _silu(t1) @ Wr2 + br2
    Etot = jnp.sum(Enode)

    Gt1 = _dsilu(t1) * Wr2[:, 0][None, :]
    Gscal = Gt1 @ Wr1.T
    Gxhat = Gscal * ln_g[None, :]
    Gh3 = (Gxhat - jnp.mean(Gxhat, axis=1, keepdims=True)
           - xhat * jnp.mean(Gxhat * xhat, axis=1, keepdims=True)) / std
    Gattn = Gh3 @ Wo.T
    P = Gattn / denom[:, None]
    beta = jnp.sum(P * attn, axis=1)
    beta_d = beta[dst]
    Pd = _sc_gather(P, dst, 200)
    ds = a * (jnp.sum(Pd * vs, axis=1) - beta_d)
    Gv, Gq, Gk = _sc_scatter128(
        [a[:, None] * Pd, (ds * inv_sqrt_h)[:, None] * ks,
         (ds * inv_sqrt_h)[:, None] * qd],
        [src, dst, src])
    sig = jax.nn.sigmoid(sig_arg)
    glen = -ds * (decay + sig * dw * edge_len)
    G2 = Gh3 + Gq @ Wq.T + Gk @ Wk.T + Gv @ Wv.T
    Gmd = G2 @ W_mp.T
    Q = Gmd / deg[:, None]
    gamma = -jnp.sum(Q * md, axis=1)
    gamma_d = gamma[dst]
    Qd = _sc_gather(Q, dst, 200)
    gh1s, = _sc_scatter128([w[:, None] * Qd], [src])
    Gh1 = G2 + gh1s
    gw_e = jnp.sum(Qd * h1s, axis=1) + gamma_d
    glen = glen + gw_e * (-_SHARP / _R_MAX) * w
    Gpre1 = Gh1 * _dsilu(pre1)
    Gagg = Gpre1 @ W_ace.T
    D = _sc_gather(Gagg, dst, 200) * h0s
    grb = D @ W_rad.T
    glen = glen + jnp.sum(grb * drb, axis=1)
    gscale = _gvec_scale(glen, len_raw)
    gvec = gscale[:, None] * edge_vec
    gpos = jax.ops.segment_sum(gvec, src, num_segments=_N) \
        - jax.ops.segment_sum(gvec, dst, num_segments=_N)
    F = -gpos

    mean_edge = jnp.mean(edge_len)
    gate = jax.nn.sigmoid(mean_edge * gw[0, 0] + gb[0])
    aux_force = gate * (_silu(scalars @ afW1 + afb1) @ afW2 + afb2)
    pooled = jnp.mean(scalars, axis=0, keepdims=True)
    stress_voigt = (_silu(pooled @ asW1 + asb1) @ asW2 + asb2).reshape(-1)
    S = jnp.zeros((3, 3), dtype=jnp.float32)
    return (Etot, F, S, aux_force, stress_voigt)
